# baseline (speedup 1.0000x reference)
import functools

import jax
import jax.numpy as jnp
import numpy as np
from jax import lax
from jax.experimental import pallas as pl
from jax.experimental.pallas import tpu as pltpu
from jax.experimental.pallas import tpu_sc as plsc

NC = 2
NS = 16
NW = NC * NS
LN = 16

_mesh = plsc.VectorSubcoreMesh(core_axis_name="c", subcore_axis_name="s")


CH = 112
GP = CH // LN


def _make_prep(N, E, R):
    EW = E // NW
    NCH = (EW + CH - 1) // CH
    FULL = EW // CH
    REMG = (EW - FULL * CH) // LN
    NPAD = ((N + 255) // 256) * 256
    RNP = R * NPAD
    DEG_LEN = ((RNP + 128 + NS * LN - 1) // (NS * LN)) * (NS * LN)
    STRIPE = DEG_LEN // NS

    @functools.partial(
        pl.kernel,
        out_type=[
            jax.ShapeDtypeStruct((NW, NCH, 3, CH), jnp.int32),
            jax.ShapeDtypeStruct((NC * DEG_LEN,), jnp.float32),
        ],
        mesh=_mesh,
        scratch_types=[
            pltpu.VMEM((EW,), jnp.int32),
            pltpu.VMEM((EW,), jnp.int32),
            pltpu.VMEM((EW,), jnp.int32),
            pltpu.VMEM((NCH, 3, CH), jnp.int32),
            pltpu.VMEM((CH,), jnp.float32),
            pltpu.VMEM((STRIPE,), jnp.float32),
            pltpu.VMEM_SHARED((DEG_LEN,), jnp.float32),
        ],
    )
    def prep(src_h, dst_h, typ_h, pk_out, deg_out,
             src_v, dst_v, typ_v, pk, ones_v, zdeg, deg_sp):
        cid = lax.axis_index("c")
        sid = lax.axis_index("s")
        w = cid * NS + sid
        base = w * EW
        pltpu.sync_copy(src_h.at[pl.ds(base, EW)], src_v)
        pltpu.sync_copy(dst_h.at[pl.ds(base, EW)], dst_v)
        pltpu.sync_copy(typ_h.at[pl.ds(base, EW)], typ_v)

        def do_group(j, k):
            e = j * CH + k * LN
            s_ = src_v[pl.ds(e, LN)]
            d_ = dst_v[pl.ds(e, LN)]
            t_ = typ_v[pl.ds(e, LN)]
            pk[j, 0, pl.ds(k * LN, LN)] = t_ * N + s_
            pk[j, 1, pl.ds(k * LN, LN)] = d_
            pk[j, 2, pl.ds(k * LN, LN)] = t_ * NPAD + d_

        @pl.loop(0, FULL)
        def _(j):
            for k in range(GP):
                do_group(j, k)

        if REMG:
            jt = FULL
            for k in range(REMG):
                do_group(jt, k)
            for k in range(REMG, GP):
                pk[jt, 0, pl.ds(k * LN, LN)] = jnp.zeros((LN,), jnp.int32)
                pk[jt, 1, pl.ds(k * LN, LN)] = jnp.full((LN,), N, jnp.int32)
                pk[jt, 2, pl.ds(k * LN, LN)] = jnp.full((LN,), RNP, jnp.int32)

        pltpu.sync_copy(pk, pk_out.at[w])

        @pl.loop(0, STRIPE // LN)
        def _(i):
            zdeg[pl.ds(i * LN, LN)] = jnp.zeros((LN,), jnp.float32)

        pltpu.sync_copy(zdeg, deg_sp.at[pl.ds(sid * STRIPE, STRIPE)])
        for k in range(GP):
            ones_v[pl.ds(k * LN, LN)] = jnp.ones((LN,), jnp.float32)
        plsc.subcore_barrier()

        @pl.loop(0, NCH)
        def _(j):
            pltpu.sync_copy(ones_v, deg_sp.at[pk.at[j, 2]], add=True)

        plsc.subcore_barrier()
        pltpu.sync_copy(deg_sp.at[pl.ds(sid * STRIPE, STRIPE)], zdeg)
        pltpu.sync_copy(zdeg,
                        deg_out.at[pl.ds(cid * DEG_LEN + sid * STRIPE, STRIPE)])

    return prep, NPAD, DEG_LEN, NCH


def _make_scale(DEG_LEN, NCH):
    @functools.partial(
        pl.kernel,
        out_type=jax.ShapeDtypeStruct((NW, NCH, CH), jnp.float32),
        mesh=_mesh,
        scratch_types=[
            pltpu.VMEM((NCH, 3, CH), jnp.int32),
            pltpu.VMEM((NCH, CH), jnp.float32),
            pltpu.SemaphoreType.DMA,
        ],
    )
    def scale(inv_h, pk_h, sc_out, buf, sbuf, sem):
        cid = lax.axis_index("c")
        sid = lax.axis_index("s")
        w = cid * NS + sid
        pltpu.sync_copy(pk_h.at[w], buf)

        @pl.loop(0, NCH)
        def _(j):
            pltpu.async_copy(inv_h.at[buf.at[j, 2]], sbuf.at[j], sem)

        @pl.loop(0, NCH)
        def _(j):
            pltpu.make_async_copy(inv_h.at[buf.at[0, 2]], sbuf.at[0],
                                  sem).wait()

        pltpu.sync_copy(sbuf, sc_out.at[w])

    return scale


def _splat_lane(v16, l):
    return lax.gather(
        v16, jnp.full((LN, 1), l, jnp.int32),
        lax.GatherDimensionNumbers(
            offset_dims=(), collapsed_slice_dims=(0,), start_index_map=(0,)),
        (1,), mode=lax.GatherScatterMode.PROMISE_IN_BOUNDS)


def _make_agg(N, H, R, NCH):
    ACC_N = ((N + 8 + 255) // 256) * 256
    ZROWS = ACC_N // NS

    @functools.partial(
        pl.kernel,
        out_type=jax.ShapeDtypeStruct((NC, ACC_N, H), jnp.float32),
        mesh=_mesh,
        scratch_types=[
            pltpu.VMEM((CH, H), jnp.float32),
            pltpu.VMEM((CH, H), jnp.float32),
            pltpu.VMEM((CH, H), jnp.float32),
            pltpu.VMEM((3, CH), jnp.int32),
            pltpu.VMEM((3, CH), jnp.int32),
            pltpu.VMEM((3, CH), jnp.int32),
            pltpu.VMEM((3, CH), jnp.int32),
            pltpu.VMEM((4, CH), jnp.float32),
            pltpu.VMEM_SHARED((ACC_N, H), jnp.float32),
            [pltpu.SemaphoreType.DMA] * 4,
            [pltpu.SemaphoreType.DMA] * 3,
            [pltpu.SemaphoreType.DMA] * 3,
        ],
    )
    def agg(t_h, pk_h, sc_h, p_out,
            r0, r1, r2, i0, i1, i2, i3, scb, acc_sp, isem, gsem, ssem):
        cid = lax.axis_index("c")
        sid = lax.axis_index("s")
        w = cid * NS + sid
        rbuf = (r0, r1, r2)
        ibuf = (i0, i1, i2, i3)

        @pl.loop(0, 16)
        def _(j):
            for k in range(H // LN):
                r0[j, pl.ds(k * LN, LN)] = jnp.zeros((LN,), jnp.float32)

        @pl.loop(0, ZROWS // 16)
        def _(j):
            pltpu.sync_copy(r0.at[pl.ds(0, 16)],
                            acc_sp.at[pl.ds(sid * ZROWS + j * 16, 16)])

        plsc.subcore_barrier()

        def i_start(c, j4):
            pltpu.async_copy(pk_h.at[w, c], ibuf[j4], isem[j4])
            pltpu.async_copy(sc_h.at[w, c], scb.at[j4], isem[j4])

        def i_wait(j4):
            pltpu.make_async_copy(pk_h.at[w, 0], ibuf[j4], isem[j4]).wait()
            pltpu.make_async_copy(sc_h.at[w, 0], scb.at[j4], isem[j4]).wait()

        HC = CH // 2

        def g_start(j3, j4):
            ib = ibuf[j4]
            rb = rbuf[j3]
            pltpu.async_copy(t_h.at[ib.at[0, pl.ds(0, HC)]],
                             rb.at[pl.ds(0, HC)], gsem[j3])
            pltpu.async_copy(t_h.at[ib.at[0, pl.ds(HC, HC)]],
                             rb.at[pl.ds(HC, HC)], gsem[j3])

        def g_wait(j3):
            pltpu.make_async_copy(t_h.at[i0.at[0, pl.ds(0, HC)]],
                                  rbuf[j3].at[pl.ds(0, HC)], gsem[j3]).wait()
            pltpu.make_async_copy(t_h.at[i0.at[0, pl.ds(0, HC)]],
                                  rbuf[j3].at[pl.ds(HC, HC)], gsem[j3]).wait()

        def s_start(j3, j4):
            pltpu.async_copy(rbuf[j3], acc_sp.at[ibuf[j4].at[1]], ssem[j3],
                             add=True)

        def s_wait(j3):
            pltpu.make_async_copy(rbuf[j3], acc_sp.at[i0.at[1]],
                                  ssem[j3]).wait()

        def scale_chunk(j3, j4):
            rb = rbuf[j3]

            @pl.loop(0, GP)
            def _(g):
                s16 = scb[j4, pl.ds(g * LN, LN)]
                for l in range(LN):
                    i = g * LN + l
                    s = _splat_lane(s16, l)
                    for k in range(H // LN):
                        rb[i, pl.ds(k * LN, LN)] = rb[i, pl.ds(k * LN, LN)] * s

        def turn(c, static):
            if static:
                j3, j4 = c % 3, c % 4
            else:
                j3, j4 = c[1] % 3, c[1] % 4
                c = c[0]
            g_wait(j3)
            scale_chunk(j3, j4)
            s_start(j3, j4)
            if static:
                if c > 0:
                    s_wait((c - 1) % 3)
                if c + 3 < NCH:
                    i_start(c + 3, (c + 3) % 4)
                if c + 2 < NCH:
                    i_wait((c + 2) % 4)
                    g_start((c + 2) % 3, (c + 2) % 4)
            else:
                s_wait((j3 + 2) % 3)

                @pl.when(c + 3 < NCH)
                def _():
                    i_start(c + 3, (j4 + 3) % 4)

                @pl.when(c + 2 < NCH)
                def _():
                    i_wait((j4 + 2) % 4)
                    g_start((j3 + 2) % 3, (j4 + 2) % 4)

        i_start(0, 0)
        i_start(1, 1)
        i_start(2, 2)
        i_wait(0)
        g_start(0, 0)
        i_wait(1)
        g_start(1, 1)

        NPEEL = min(6, NCH)
        for c in range(NPEEL):
            turn(c, True)
        NBLK = (NCH - NPEEL) // 12

        @pl.loop(0, NBLK)
        def _(p):
            for u in range(12):
                turn((NPEEL + p * 12 + u, NPEEL + u), False)

        for c in range(NPEEL + NBLK * 12, NCH):
            turn(c, True)

        if NCH > 0:
            s_wait((NCH - 1) % 3)
        plsc.subcore_barrier()

        @pl.loop(0, ZROWS // 128)
        def _(j):
            rr = sid * ZROWS + j * 128
            pltpu.sync_copy(acc_sp.at[pl.ds(rr, 128)],
                            p_out.at[cid, pl.ds(rr, 128)])

    return agg, ACC_N


def _inv_deg(deg):
    dl = deg.shape[0] // NC
    d3 = deg.reshape(NC, dl // 128, 128)

    def body(d_ref, o_ref):
        o_ref[...] = 1.0 / jnp.maximum(d_ref[0] + d_ref[1], 1.0)

    out = pl.pallas_call(
        body, out_shape=jax.ShapeDtypeStruct((dl // 128, 128), jnp.float32)
    )(d3)
    return out.reshape(dl)


def _mm_first(h, W, loopW, BN):
    N, D = h.shape
    R, _, H = W.shape

    def body(h_ref, w_ref, lw_ref, t_ref, s_ref):
        hb = h_ref[...]
        for r in range(R):
            t_ref[r] = jnp.dot(hb, w_ref[r], preferred_element_type=jnp.float32)
        s_ref[...] = jnp.dot(hb, lw_ref[...], preferred_element_type=jnp.float32)

    grid = (N // BN,)
    T, S = pl.pallas_call(
        body,
        grid=grid,
        in_specs=[
            pl.BlockSpec((BN, D), lambda i: (i, 0)),
            pl.BlockSpec((R, D, H), lambda i: (0, 0, 0)),
            pl.BlockSpec((D, H), lambda i: (0, 0)),
        ],
        out_specs=[
            pl.BlockSpec((R, BN, H), lambda i: (0, i, 0)),
            pl.BlockSpec((BN, H), lambda i: (i, 0)),
        ],
        out_shape=[
            jax.ShapeDtypeStruct((R, N, H), jnp.float32),
            jax.ShapeDtypeStruct((N, H), jnp.float32),
        ],
    )(h, W, loopW)
    return T, S


def _mm_second(S1, P1, b1, W, loopW, BN):
    N, H0 = S1.shape
    R, _, H = W.shape

    def body(s1_ref, p_ref, b_ref, w_ref, lw_ref, t_ref, s_ref):
        hb = jax.nn.relu(s1_ref[...] + p_ref[0] + p_ref[1] + b_ref[...])
        for r in range(R):
            t_ref[r] = jnp.dot(hb, w_ref[r], preferred_element_type=jnp.float32)
        s_ref[...] = jnp.dot(hb, lw_ref[...], preferred_element_type=jnp.float32)

    grid = (N // BN,)
    T, S = pl.pallas_call(
        body,
        grid=grid,
        in_specs=[
            pl.BlockSpec((BN, H0), lambda i: (i, 0)),
            pl.BlockSpec((NC, BN, H0), lambda i: (0, i, 0)),
            pl.BlockSpec((H0,), lambda i: (0,)),
            pl.BlockSpec((R, H0, H), lambda i: (0, 0, 0)),
            pl.BlockSpec((H0, H), lambda i: (0, 0)),
        ],
        out_specs=[
            pl.BlockSpec((R, BN, H), lambda i: (0, i, 0)),
            pl.BlockSpec((BN, H), lambda i: (i, 0)),
        ],
        out_shape=[
            jax.ShapeDtypeStruct((R, N, H), jnp.float32),
            jax.ShapeDtypeStruct((N, H), jnp.float32),
        ],
    )(S1, P1, b1, W, loopW)
    return T, S


def _mlp_head(S2, P2, b2, x, oW1a, oW1b, ob1, oW2, ob2, BN):
    N, H = S2.shape
    D = x.shape[1]
    C = oW2.shape[1]

    def body(s2_ref, p_ref, b_ref, x_ref, wa_ref, wb_ref, b1_ref, w2_ref,
             b2_ref, o_ref):
        h2 = jax.nn.relu(s2_ref[...] + p_ref[0] + p_ref[1] + b_ref[...])
        z = jax.nn.relu(
            jnp.dot(h2, wa_ref[...], preferred_element_type=jnp.float32)
            + jnp.dot(x_ref[...], wb_ref[...], preferred_element_type=jnp.float32)
            + b1_ref[...])
        o_ref[...] = (
            jnp.dot(z, w2_ref[...], preferred_element_type=jnp.float32)
            + b2_ref[...])

    grid = (N // BN,)
    return pl.pallas_call(
        body,
        grid=grid,
        in_specs=[
            pl.BlockSpec((BN, H), lambda i: (i, 0)),
            pl.BlockSpec((NC, BN, H), lambda i: (0, i, 0)),
            pl.BlockSpec((H,), lambda i: (0,)),
            pl.BlockSpec((BN, D), lambda i: (i, 0)),
            pl.BlockSpec((H, H), lambda i: (0, 0)),
            pl.BlockSpec((D, H), lambda i: (0, 0)),
            pl.BlockSpec((H,), lambda i: (0,)),
            pl.BlockSpec((H, C), lambda i: (0, 0)),
            pl.BlockSpec((C,), lambda i: (0,)),
        ],
        out_specs=pl.BlockSpec((BN, C), lambda i: (i, 0)),
        out_shape=jax.ShapeDtypeStruct((N, C), jnp.float32),
    )(S2, P2, b2, x, oW1a, oW1b, ob1, oW2, ob2)


@jax.jit
def kernel(x, edge_index, edge_type, W1, loopW1, b1, W2, loopW2, b2,
           oW1, ob1, oW2, ob2):
    N, D = x.shape
    E = edge_index.shape[1]
    R, _, H = W1.shape

    prep, NPAD, DEG_LEN, NCH = _make_prep(N, E, R)
    scale = _make_scale(DEG_LEN, NCH)
    agg, ACC_N = _make_agg(N, H, R, NCH)
    BN = 2000

    src = edge_index[0]
    dst = edge_index[1]
    pk, deg = prep(src, dst, edge_type)
    inv = _inv_deg(deg)
    scv = scale(inv, pk)

    T1, S1 = _mm_first(x, W1, loopW1, BN)
    P1 = agg(T1.reshape(R * N, H), pk, scv)
    T2, S2 = _mm_second(S1, P1, b1, W2, loopW2, BN)
    P2 = agg(T2.reshape(R * N, H), pk, scv)
    return _mlp_head(S2, P2, b2, x, oW1[:H], oW1[H:], ob1, oW2, ob2, BN)


# speedup vs baseline: 1.0590x; 1.0590x over previous
import functools

import jax
import jax.numpy as jnp
import numpy as np
from jax import lax
from jax.experimental import pallas as pl
from jax.experimental.pallas import tpu as pltpu
from jax.experimental.pallas import tpu_sc as plsc

NC = 2
NS = 16
NW = NC * NS
LN = 16

_mesh = plsc.VectorSubcoreMesh(core_axis_name="c", subcore_axis_name="s")


CH = 112
GP = CH // LN


def _make_prep(N, E, R):
    EW = E // NW
    NCH = (EW + CH - 1) // CH
    FULL = EW // CH
    REMG = (EW - FULL * CH) // LN
    NPAD = ((N + 255) // 256) * 256
    RNP = R * NPAD
    DEG_LEN = ((RNP + 128 + NS * LN - 1) // (NS * LN)) * (NS * LN)
    STRIPE = DEG_LEN // NS

    @functools.partial(
        pl.kernel,
        out_type=[
            jax.ShapeDtypeStruct((NW, NCH, 3, CH), jnp.int32),
            jax.ShapeDtypeStruct((NC * DEG_LEN,), jnp.float32),
        ],
        mesh=_mesh,
        scratch_types=[
            pltpu.VMEM((EW,), jnp.int32),
            pltpu.VMEM((EW,), jnp.int32),
            pltpu.VMEM((EW,), jnp.int32),
            pltpu.VMEM((NCH, 3, CH), jnp.int32),
            pltpu.VMEM((CH,), jnp.float32),
            pltpu.VMEM((STRIPE,), jnp.float32),
            pltpu.VMEM_SHARED((DEG_LEN,), jnp.float32),
        ],
    )
    def prep(src_h, dst_h, typ_h, pk_out, deg_out,
             src_v, dst_v, typ_v, pk, ones_v, zdeg, deg_sp):
        cid = lax.axis_index("c")
        sid = lax.axis_index("s")
        w = cid * NS + sid
        base = w * EW
        pltpu.sync_copy(src_h.at[pl.ds(base, EW)], src_v)
        pltpu.sync_copy(dst_h.at[pl.ds(base, EW)], dst_v)
        pltpu.sync_copy(typ_h.at[pl.ds(base, EW)], typ_v)

        def do_group(j, k):
            e = j * CH + k * LN
            s_ = src_v[pl.ds(e, LN)]
            d_ = dst_v[pl.ds(e, LN)]
            t_ = typ_v[pl.ds(e, LN)]
            pk[j, 0, pl.ds(k * LN, LN)] = t_ * N + s_
            pk[j, 1, pl.ds(k * LN, LN)] = d_
            pk[j, 2, pl.ds(k * LN, LN)] = t_ * NPAD + d_

        @pl.loop(0, FULL)
        def _(j):
            for k in range(GP):
                do_group(j, k)

        if REMG:
            jt = FULL
            for k in range(REMG):
                do_group(jt, k)
            for k in range(REMG, GP):
                pk[jt, 0, pl.ds(k * LN, LN)] = jnp.zeros((LN,), jnp.int32)
                pk[jt, 1, pl.ds(k * LN, LN)] = jnp.full((LN,), N, jnp.int32)
                pk[jt, 2, pl.ds(k * LN, LN)] = jnp.full((LN,), RNP, jnp.int32)

        pltpu.sync_copy(pk, pk_out.at[w])

        @pl.loop(0, STRIPE // LN)
        def _(i):
            zdeg[pl.ds(i * LN, LN)] = jnp.zeros((LN,), jnp.float32)

        pltpu.sync_copy(zdeg, deg_sp.at[pl.ds(sid * STRIPE, STRIPE)])
        for k in range(GP):
            ones_v[pl.ds(k * LN, LN)] = jnp.ones((LN,), jnp.float32)
        plsc.subcore_barrier()

        @pl.loop(0, NCH)
        def _(j):
            pltpu.sync_copy(ones_v, deg_sp.at[pk.at[j, 2]], add=True)

        plsc.subcore_barrier()
        pltpu.sync_copy(deg_sp.at[pl.ds(sid * STRIPE, STRIPE)], zdeg)
        pltpu.sync_copy(zdeg,
                        deg_out.at[pl.ds(cid * DEG_LEN + sid * STRIPE, STRIPE)])

    return prep, NPAD, DEG_LEN, NCH


def _splat_lane(v16, l):
    return lax.gather(
        v16, jnp.full((LN, 1), l, jnp.int32),
        lax.GatherDimensionNumbers(
            offset_dims=(), collapsed_slice_dims=(0,), start_index_map=(0,)),
        (1,), mode=lax.GatherScatterMode.PROMISE_IN_BOUNDS)


def _make_agg(N, H, R, NCH):
    ACC_N = ((N + 8 + 255) // 256) * 256
    ZROWS = ACC_N // NS

    @functools.partial(
        pl.kernel,
        out_type=jax.ShapeDtypeStruct((NC, ACC_N, H), jnp.float32),
        mesh=_mesh,
        scratch_types=[
            pltpu.VMEM((CH, H), jnp.float32),
            pltpu.VMEM((CH, H), jnp.float32),
            pltpu.VMEM((CH, H), jnp.float32),
            pltpu.VMEM((3, CH), jnp.int32),
            pltpu.VMEM((3, CH), jnp.int32),
            pltpu.VMEM((3, CH), jnp.int32),
            pltpu.VMEM((3, CH), jnp.int32),
            pltpu.VMEM((4, CH), jnp.float32),
            pltpu.VMEM_SHARED((ACC_N, H), jnp.float32),
            [pltpu.SemaphoreType.DMA] * 4,
            [pltpu.SemaphoreType.DMA] * 4,
            [pltpu.SemaphoreType.DMA] * 3,
            [pltpu.SemaphoreType.DMA] * 3,
        ],
    )
    def agg(t_h, pk_h, inv_h, p_out,
            r0, r1, r2, i0, i1, i2, i3, scb, acc_sp, isem, vsem, gsem, ssem):
        cid = lax.axis_index("c")
        sid = lax.axis_index("s")
        w = cid * NS + sid
        rbuf = (r0, r1, r2)
        ibuf = (i0, i1, i2, i3)

        @pl.loop(0, 16)
        def _(j):
            for k in range(H // LN):
                r0[j, pl.ds(k * LN, LN)] = jnp.zeros((LN,), jnp.float32)

        @pl.loop(0, ZROWS // 16)
        def _(j):
            pltpu.sync_copy(r0.at[pl.ds(0, 16)],
                            acc_sp.at[pl.ds(sid * ZROWS + j * 16, 16)])

        plsc.subcore_barrier()

        def i_start(c, j4):
            pltpu.async_copy(pk_h.at[w, c], ibuf[j4], isem[j4])

        def i_wait(j4):
            pltpu.make_async_copy(pk_h.at[w, 0], ibuf[j4], isem[j4]).wait()

        def v_start(j4):
            pltpu.async_copy(inv_h.at[ibuf[j4].at[2]], scb.at[j4], vsem[j4])

        def v_wait(j4):
            pltpu.make_async_copy(inv_h.at[i0.at[2]], scb.at[j4],
                                  vsem[j4]).wait()

        HC = CH // 2

        def g_start(j3, j4):
            ib = ibuf[j4]
            rb = rbuf[j3]
            pltpu.async_copy(t_h.at[ib.at[0, pl.ds(0, HC)]],
                             rb.at[pl.ds(0, HC)], gsem[j3])
            pltpu.async_copy(t_h.at[ib.at[0, pl.ds(HC, HC)]],
                             rb.at[pl.ds(HC, HC)], gsem[j3])

        def g_wait(j3):
            pltpu.make_async_copy(t_h.at[i0.at[0, pl.ds(0, HC)]],
                                  rbuf[j3].at[pl.ds(0, HC)], gsem[j3]).wait()
            pltpu.make_async_copy(t_h.at[i0.at[0, pl.ds(0, HC)]],
                                  rbuf[j3].at[pl.ds(HC, HC)], gsem[j3]).wait()

        def s_start(j3, j4):
            pltpu.async_copy(rbuf[j3], acc_sp.at[ibuf[j4].at[1]], ssem[j3],
                             add=True)

        def s_wait(j3):
            pltpu.make_async_copy(rbuf[j3], acc_sp.at[i0.at[1]],
                                  ssem[j3]).wait()

        def scale_chunk(j3, j4):
            rb = rbuf[j3]

            @pl.loop(0, GP)
            def _(g):
                s16 = scb[j4, pl.ds(g * LN, LN)]
                for l in range(LN):
                    i = g * LN + l
                    s = _splat_lane(s16, l)
                    for k in range(H // LN):
                        rb[i, pl.ds(k * LN, LN)] = rb[i, pl.ds(k * LN, LN)] * s

        def turn(c, static):
            if static:
                j3, j4 = c % 3, c % 4
            else:
                j3, j4 = c[1] % 3, c[1] % 4
                c = c[0]
            g_wait(j3)
            v_wait(j4)
            scale_chunk(j3, j4)
            s_start(j3, j4)
            if static:
                if c > 0:
                    s_wait((c - 1) % 3)
                if c + 3 < NCH:
                    i_start(c + 3, (c + 3) % 4)
                if c + 2 < NCH:
                    i_wait((c + 2) % 4)
                    v_start((c + 2) % 4)
                    g_start((c + 2) % 3, (c + 2) % 4)
            else:
                s_wait((j3 + 2) % 3)

                @pl.when(c + 3 < NCH)
                def _():
                    i_start(c + 3, (j4 + 3) % 4)

                @pl.when(c + 2 < NCH)
                def _():
                    i_wait((j4 + 2) % 4)
                    v_start((j4 + 2) % 4)
                    g_start((j3 + 2) % 3, (j4 + 2) % 4)

        i_start(0, 0)
        i_start(1, 1)
        i_start(2, 2)
        i_wait(0)
        v_start(0)
        g_start(0, 0)
        i_wait(1)
        v_start(1)
        g_start(1, 1)

        NPEEL = min(6, NCH)
        for c in range(NPEEL):
            turn(c, True)
        NBLK = (NCH - NPEEL) // 12

        @pl.loop(0, NBLK)
        def _(p):
            for u in range(12):
                turn((NPEEL + p * 12 + u, NPEEL + u), False)

        for c in range(NPEEL + NBLK * 12, NCH):
            turn(c, True)

        if NCH > 0:
            s_wait((NCH - 1) % 3)
        plsc.subcore_barrier()

        @pl.loop(0, ZROWS // 128)
        def _(j):
            rr = sid * ZROWS + j * 128
            pltpu.sync_copy(acc_sp.at[pl.ds(rr, 128)],
                            p_out.at[cid, pl.ds(rr, 128)])

    return agg, ACC_N


def _inv_deg(deg):
    dl = deg.shape[0] // NC
    d3 = deg.reshape(NC, dl // 128, 128)

    def body(d_ref, o_ref):
        o_ref[...] = 1.0 / jnp.maximum(d_ref[0] + d_ref[1], 1.0)

    out = pl.pallas_call(
        body, out_shape=jax.ShapeDtypeStruct((dl // 128, 128), jnp.float32)
    )(d3)
    return out.reshape(dl)


def _mm_first(h, W, loopW, BN):
    N, D = h.shape
    R, _, H = W.shape

    def body(h_ref, w_ref, lw_ref, t_ref, s_ref):
        hb = h_ref[...]
        for r in range(R):
            t_ref[r] = jnp.dot(hb, w_ref[r], preferred_element_type=jnp.float32)
        s_ref[...] = jnp.dot(hb, lw_ref[...], preferred_element_type=jnp.float32)

    grid = (N // BN,)
    T, S = pl.pallas_call(
        body,
        grid=grid,
        in_specs=[
            pl.BlockSpec((BN, D), lambda i: (i, 0)),
            pl.BlockSpec((R, D, H), lambda i: (0, 0, 0)),
            pl.BlockSpec((D, H), lambda i: (0, 0)),
        ],
        out_specs=[
            pl.BlockSpec((R, BN, H), lambda i: (0, i, 0)),
            pl.BlockSpec((BN, H), lambda i: (i, 0)),
        ],
        out_shape=[
            jax.ShapeDtypeStruct((R, N, H), jnp.float32),
            jax.ShapeDtypeStruct((N, H), jnp.float32),
        ],
    )(h, W, loopW)
    return T, S


def _mm_second(S1, P1, b1, W, loopW, BN):
    N, H0 = S1.shape
    R, _, H = W.shape

    def body(s1_ref, p_ref, b_ref, w_ref, lw_ref, t_ref, s_ref):
        hb = jax.nn.relu(s1_ref[...] + p_ref[0] + p_ref[1] + b_ref[...])
        for r in range(R):
            t_ref[r] = jnp.dot(hb, w_ref[r], preferred_element_type=jnp.float32)
        s_ref[...] = jnp.dot(hb, lw_ref[...], preferred_element_type=jnp.float32)

    grid = (N // BN,)
    T, S = pl.pallas_call(
        body,
        grid=grid,
        in_specs=[
            pl.BlockSpec((BN, H0), lambda i: (i, 0)),
            pl.BlockSpec((NC, BN, H0), lambda i: (0, i, 0)),
            pl.BlockSpec((H0,), lambda i: (0,)),
            pl.BlockSpec((R, H0, H), lambda i: (0, 0, 0)),
            pl.BlockSpec((H0, H), lambda i: (0, 0)),
        ],
        out_specs=[
            pl.BlockSpec((R, BN, H), lambda i: (0, i, 0)),
            pl.BlockSpec((BN, H), lambda i: (i, 0)),
        ],
        out_shape=[
            jax.ShapeDtypeStruct((R, N, H), jnp.float32),
            jax.ShapeDtypeStruct((N, H), jnp.float32),
        ],
    )(S1, P1, b1, W, loopW)
    return T, S


def _mlp_head(S2, P2, b2, x, oW1a, oW1b, ob1, oW2, ob2, BN):
    N, H = S2.shape
    D = x.shape[1]
    C = oW2.shape[1]

    def body(s2_ref, p_ref, b_ref, x_ref, wa_ref, wb_ref, b1_ref, w2_ref,
             b2_ref, o_ref):
        h2 = jax.nn.relu(s2_ref[...] + p_ref[0] + p_ref[1] + b_ref[...])
        z = jax.nn.relu(
            jnp.dot(h2, wa_ref[...], preferred_element_type=jnp.float32)
            + jnp.dot(x_ref[...], wb_ref[...], preferred_element_type=jnp.float32)
            + b1_ref[...])
        o_ref[...] = (
            jnp.dot(z, w2_ref[...], preferred_element_type=jnp.float32)
            + b2_ref[...])

    grid = (N // BN,)
    return pl.pallas_call(
        body,
        grid=grid,
        in_specs=[
            pl.BlockSpec((BN, H), lambda i: (i, 0)),
            pl.BlockSpec((NC, BN, H), lambda i: (0, i, 0)),
            pl.BlockSpec((H,), lambda i: (0,)),
            pl.BlockSpec((BN, D), lambda i: (i, 0)),
            pl.BlockSpec((H, H), lambda i: (0, 0)),
            pl.BlockSpec((D, H), lambda i: (0, 0)),
            pl.BlockSpec((H,), lambda i: (0,)),
            pl.BlockSpec((H, C), lambda i: (0, 0)),
            pl.BlockSpec((C,), lambda i: (0,)),
        ],
        out_specs=pl.BlockSpec((BN, C), lambda i: (i, 0)),
        out_shape=jax.ShapeDtypeStruct((N, C), jnp.float32),
    )(S2, P2, b2, x, oW1a, oW1b, ob1, oW2, ob2)


@jax.jit
def kernel(x, edge_index, edge_type, W1, loopW1, b1, W2, loopW2, b2,
           oW1, ob1, oW2, ob2):
    N, D = x.shape
    E = edge_index.shape[1]
    R, _, H = W1.shape

    prep, NPAD, DEG_LEN, NCH = _make_prep(N, E, R)
    agg, ACC_N = _make_agg(N, H, R, NCH)
    BN = 2000

    src = edge_index[0]
    dst = edge_index[1]
    pk, deg = prep(src, dst, edge_type)
    inv = _inv_deg(deg)

    T1, S1 = _mm_first(x, W1, loopW1, BN)
    P1 = agg(T1.reshape(R * N, H), pk, inv)
    T2, S2 = _mm_second(S1, P1, b1, W2, loopW2, BN)
    P2 = agg(T2.reshape(R * N, H), pk, inv)
    return _mlp_head(S2, P2, b2, x, oW1[:H], oW1[H:], ob1, oW2, ob2, BN)
